# Initial kernel scaffold; baseline (speedup 1.0000x reference)
import jax
import jax.numpy as jnp
from jax.experimental import pallas as pl


def kernel(x, edge_index, batch, W1, b1, W2, b2, W3, b3, Wl1, bl1, Wl2, bl2):
    raise NotImplementedError("write your pallas kernel here")


# speedup vs baseline: 19.2449x; 19.2449x over previous
import functools

import jax
import jax.numpy as jnp
from jax import lax
from jax.experimental import pallas as pl
from jax.experimental.pallas import tpu as pltpu
from jax.experimental.pallas import tpu_sc as plsc

F32 = jnp.float32
I32 = jnp.int32

_NC = 2
_NS = 16
_NW = _NC * _NS
_CHUNK = 128
_G = 64


def _sc_mesh():
    return plsc.VectorSubcoreMesh(core_axis_name="c", subcore_axis_name="s")


def _make_deg_kernel(nch, n_acc, rpt):
    @functools.partial(
        pl.kernel,
        out_type=jax.ShapeDtypeStruct((_NC * n_acc,), F32),
        mesh=_sc_mesh(),
        scratch_types=[
            pltpu.VMEM((nch, _CHUNK), I32),
            pltpu.VMEM((_CHUNK,), F32),
            pltpu.VMEM_SHARED((n_acc,), F32),
            pltpu.SemaphoreType.DMA,
        ],
    )
    def deg_kernel(dst_hbm, zeros_hbm, ones_hbm, out_hbm, dst_v, ones_v, acc, sem):
        cid = lax.axis_index("c")
        sid = lax.axis_index("s")
        wid = cid * _NS + sid
        pltpu.sync_copy(zeros_hbm, acc.at[pl.ds(sid * rpt, rpt)])
        pltpu.sync_copy(dst_hbm.at[pl.ds(wid * nch, nch)], dst_v)
        pltpu.sync_copy(ones_hbm, ones_v)
        plsc.subcore_barrier()

        def body(j, carry):
            pltpu.sync_copy(ones_v, acc.at[dst_v.at[j]], add=True)
            return carry

        lax.fori_loop(0, nch, body, 0)
        plsc.subcore_barrier()
        pltpu.sync_copy(acc.at[pl.ds(sid * rpt, rpt)],
                        out_hbm.at[pl.ds(cid * n_acc + sid * rpt, rpt)])

    return deg_kernel


def _make_agg_kernel(nch, n_acc, rpt, d):
    @functools.partial(
        pl.kernel,
        out_type=jax.ShapeDtypeStruct((_NC, n_acc, d), F32),
        mesh=_sc_mesh(),
        scratch_types=[
            pltpu.VMEM((nch, _CHUNK), I32),
            pltpu.VMEM((nch, _CHUNK), I32),
            pltpu.VMEM((_CHUNK, d), F32),
            pltpu.VMEM_SHARED((n_acc, d), F32),
            pltpu.SemaphoreType.DMA,
        ],
    )
    def agg_kernel(y_hbm, src_hbm, dst_hbm, zeros_hbm, out_hbm,
                   src_v, dst_v, rows_v, acc, sem):
        cid = lax.axis_index("c")
        sid = lax.axis_index("s")
        wid = cid * _NS + sid
        pltpu.sync_copy(zeros_hbm, acc.at[pl.ds(sid * rpt, rpt)])
        pltpu.sync_copy(src_hbm.at[pl.ds(wid * nch, nch)], src_v)
        pltpu.sync_copy(dst_hbm.at[pl.ds(wid * nch, nch)], dst_v)
        plsc.subcore_barrier()

        def body(j, carry):
            pltpu.async_copy(y_hbm.at[src_v.at[j]], rows_v, sem).wait()
            pltpu.sync_copy(rows_v, acc.at[dst_v.at[j]], add=True)
            return carry

        lax.fori_loop(0, nch, body, 0)
        plsc.subcore_barrier()
        pltpu.sync_copy(acc.at[pl.ds(sid * rpt, rpt)],
                        out_hbm.at[cid, pl.ds(sid * rpt, rpt)])

    return agg_kernel


def kernel(x, edge_index, batch, W1, b1, W2, b2, W3, b3, Wl1, bl1, Wl2, bl2):
    n, d = x.shape
    e = edge_index.shape[1]

    rpt = ((-(-(n + 1) // _NS)) + 127) // 128 * 128
    n_acc = rpt * _NS
    epc = _NW * _CHUNK
    nch = (-(-e // epc) + 7) // 8 * 8
    e_pad = nch * epc
    pad = e_pad - e

    src = edge_index[0]
    dst = edge_index[1]
    if pad:
        pad_i = jnp.arange(pad, dtype=I32)
        src = jnp.concatenate([src, pad_i % min(n, 1024)])
        dst = jnp.concatenate([dst, n + pad_i % (n_acc - n)])
    src2 = src.reshape(_NW * nch, _CHUNK)
    dst2 = dst.reshape(_NW * nch, _CHUNK)

    zeros_r = jnp.zeros((rpt,), F32)
    zeros_rd = jnp.zeros((rpt, d), F32)
    ones_c = jnp.ones((_CHUNK,), F32)

    deg_k = _make_deg_kernel(nch, n_acc, rpt)
    agg_k = _make_agg_kernel(nch, n_acc, rpt, d)

    deg = deg_k(dst2, zeros_r, ones_c)
    d0 = deg[:n].reshape(n, 1)
    d1 = deg[n_acc:n_acc + n].reshape(n, 1)

    def t1_body(x_ref, w_ref, d0_ref, d1_ref, y_ref, dinv_ref):
        dinv = lax.rsqrt(1.0 + d0_ref[...] + d1_ref[...])
        xw = jnp.dot(x_ref[...], w_ref[...])
        y_ref[...] = dinv * xw
        dinv_ref[...] = dinv

    y, dinv = pl.pallas_call(
        t1_body,
        out_shape=(jax.ShapeDtypeStruct((n, d), F32),
                   jax.ShapeDtypeStruct((n, 1), F32)),
    )(x, W1, d0, d1)

    def t2_body(z_ref, y_ref, dinv_ref, b_ref, w_ref, yn_ref):
        dinv = dinv_ref[...]
        agg = z_ref[0, 0:n, :] + z_ref[1, 0:n, :] + y_ref[...]
        h = jnp.maximum(dinv * agg + b_ref[...], 0.0)
        yn_ref[...] = dinv * jnp.dot(h, w_ref[...])

    t2 = pl.pallas_call(t2_body, out_shape=jax.ShapeDtypeStruct((n, d), F32))

    z = agg_k(y, src2, dst2, zeros_rd)
    y = t2(z, y, dinv, b1.reshape(1, d), W2)
    z = agg_k(y, src2, dst2, zeros_rd)
    y = t2(z, y, dinv, b2.reshape(1, d), W3)
    z = agg_k(y, src2, dst2, zeros_rd)

    def t3_body(z_ref, y_ref, dinv_ref, b_ref, batch_ref,
                wl1_ref, bl1_ref, wl2_ref, bl2_ref, out_ref):
        dinv = dinv_ref[...]
        h3 = dinv * (z_ref[0, 0:n, :] + z_ref[1, 0:n, :] + y_ref[...]) + b_ref[...]
        gids = lax.broadcasted_iota(I32, (_G, n), 0)
        oh = (batch_ref[...] == gids).astype(F32)
        sums = jnp.dot(oh, h3)
        cnts = jnp.sum(oh, axis=1, keepdims=True)
        pooled = sums / jnp.maximum(cnts, 1.0)
        h2 = jnp.maximum(
            jnp.dot(pooled, wl1_ref[...]) + bl1_ref[...], 0.0)
        out_ref[...] = jnp.dot(h2, wl2_ref[...]) + bl2_ref[...]

    out = pl.pallas_call(
        t3_body, out_shape=jax.ShapeDtypeStruct((_G, 1), F32),
    )(z, y, dinv, b3.reshape(1, d), batch.reshape(1, n),
      Wl1, bl1.reshape(1, -1), Wl2, bl2.reshape(1, 1))
    return out


# speedup vs baseline: 28.1306x; 1.4617x over previous
import functools

import jax
import jax.numpy as jnp
from jax import lax
from jax.experimental import pallas as pl
from jax.experimental.pallas import tpu as pltpu
from jax.experimental.pallas import tpu_sc as plsc

F32 = jnp.float32
I32 = jnp.int32

_NC = 2
_NS = 16
_NW = _NC * _NS
_CHUNK = 128
_G = 64


def _sc_mesh():
    return plsc.VectorSubcoreMesh(core_axis_name="c", subcore_axis_name="s")


def _make_deg_kernel(nch, n_acc, rpt):
    @functools.partial(
        pl.kernel,
        out_type=jax.ShapeDtypeStruct((_NC * n_acc,), F32),
        mesh=_sc_mesh(),
        scratch_types=[
            pltpu.VMEM((nch, _CHUNK), I32),
            pltpu.VMEM((_CHUNK,), F32),
            pltpu.VMEM_SHARED((n_acc,), F32),
            pltpu.SemaphoreType.DMA,
        ],
    )
    def deg_kernel(dst_hbm, zeros_hbm, ones_hbm, out_hbm, dst_v, ones_v, acc, sem):
        cid = lax.axis_index("c")
        sid = lax.axis_index("s")
        wid = cid * _NS + sid
        pltpu.sync_copy(zeros_hbm, acc.at[pl.ds(sid * rpt, rpt)])
        pltpu.sync_copy(dst_hbm.at[pl.ds(wid * nch, nch)], dst_v)
        pltpu.sync_copy(ones_hbm, ones_v)
        plsc.subcore_barrier()

        def body(j, carry):
            pltpu.sync_copy(ones_v, acc.at[dst_v.at[j]], add=True)
            return carry

        lax.fori_loop(0, nch, body, 0)
        plsc.subcore_barrier()
        pltpu.sync_copy(acc.at[pl.ds(sid * rpt, rpt)],
                        out_hbm.at[pl.ds(cid * n_acc + sid * rpt, rpt)])

    return deg_kernel


def _make_agg_kernel(nch, n_acc, rpt, d):
    @functools.partial(
        pl.kernel,
        out_type=jax.ShapeDtypeStruct((_NC, n_acc, d), F32),
        mesh=_sc_mesh(),
        scratch_types=[
            pltpu.VMEM((nch // 2, _CHUNK), I32),
            pltpu.VMEM((nch // 2, _CHUNK), I32),
            pltpu.VMEM((_CHUNK, d), F32),
            pltpu.VMEM((_CHUNK, d), F32),
            pltpu.VMEM_SHARED((n_acc, d), F32),
            pltpu.SemaphoreType.DMA,
            pltpu.SemaphoreType.DMA,
        ],
    )
    def agg_kernel(y_hbm, src_hbm, dst_hbm, zeros_hbm, out_hbm,
                   src_v, dst_v, rows0, rows1, acc, sem0, sem1):
        cid = lax.axis_index("c")
        sid = lax.axis_index("s")
        wid = cid * _NS + sid
        nh = nch // 2
        pltpu.sync_copy(zeros_hbm, acc.at[pl.ds(sid * rpt, rpt)])
        plsc.subcore_barrier()

        for half in range(2):
            base = wid * nch + half * nh
            pltpu.sync_copy(src_hbm.at[pl.ds(base, nh)], src_v)
            pltpu.sync_copy(dst_hbm.at[pl.ds(base, nh)], dst_v)
            pltpu.async_copy(y_hbm.at[src_v.at[0]], rows0, sem0)

            def body(jj, carry):
                j0 = 2 * jj
                j1 = j0 + 1
                pltpu.async_copy(y_hbm.at[src_v.at[j1]], rows1, sem1)
                pltpu.make_async_copy(y_hbm.at[src_v.at[j0]], rows0,
                                      sem0).wait()
                pltpu.sync_copy(rows0, acc.at[dst_v.at[j0]], add=True)

                @pl.when(j0 + 2 < nh)
                def _():
                    pltpu.async_copy(y_hbm.at[src_v.at[j0 + 2]], rows0, sem0)

                pltpu.make_async_copy(y_hbm.at[src_v.at[j1]], rows1,
                                      sem1).wait()
                pltpu.sync_copy(rows1, acc.at[dst_v.at[j1]], add=True)
                return carry

            lax.fori_loop(0, nh // 2, body, 0)
        plsc.subcore_barrier()
        pltpu.sync_copy(acc.at[pl.ds(sid * rpt, rpt)],
                        out_hbm.at[cid, pl.ds(sid * rpt, rpt)])

    return agg_kernel


def kernel(x, edge_index, batch, W1, b1, W2, b2, W3, b3, Wl1, bl1, Wl2, bl2):
    n, d = x.shape
    e = edge_index.shape[1]

    rpt = ((-(-(n + 1) // _NS)) + 127) // 128 * 128
    n_acc = rpt * _NS
    epc = _NW * _CHUNK
    nch = (-(-e // epc) + 7) // 8 * 8
    e_pad = nch * epc
    pad = e_pad - e

    src = edge_index[0]
    dst = edge_index[1]
    if pad:
        pad_i = jnp.arange(pad, dtype=I32)
        src = jnp.concatenate([src, pad_i % min(n, 1024)])
        dst = jnp.concatenate([dst, n + pad_i % (n_acc - n)])
    src2 = src.reshape(_NW * nch, _CHUNK)
    dst2 = dst.reshape(_NW * nch, _CHUNK)

    zeros_r = jnp.zeros((rpt,), F32)
    zeros_rd = jnp.zeros((rpt, d), F32)
    ones_c = jnp.ones((_CHUNK,), F32)

    deg_k = _make_deg_kernel(nch, n_acc, rpt)
    agg_k = _make_agg_kernel(nch, n_acc, rpt, d)

    deg = deg_k(dst2, zeros_r, ones_c)
    d0 = deg[:n].reshape(n, 1)
    d1 = deg[n_acc:n_acc + n].reshape(n, 1)

    def t1_body(x_ref, w_ref, d0_ref, d1_ref, y_ref, dinv_ref):
        dinv = lax.rsqrt(1.0 + d0_ref[...] + d1_ref[...])
        xw = jnp.dot(x_ref[...], w_ref[...])
        y_ref[...] = dinv * xw
        dinv_ref[...] = dinv

    y, dinv = pl.pallas_call(
        t1_body,
        out_shape=(jax.ShapeDtypeStruct((n, d), F32),
                   jax.ShapeDtypeStruct((n, 1), F32)),
    )(x, W1, d0, d1)

    def t2_body(z_ref, y_ref, dinv_ref, b_ref, w_ref, yn_ref):
        dinv = dinv_ref[...]
        agg = z_ref[0, 0:n, :] + z_ref[1, 0:n, :] + y_ref[...]
        h = jnp.maximum(dinv * agg + b_ref[...], 0.0)
        yn_ref[...] = dinv * jnp.dot(h, w_ref[...])

    t2 = pl.pallas_call(t2_body, out_shape=jax.ShapeDtypeStruct((n, d), F32))

    z = agg_k(y, src2, dst2, zeros_rd)
    y = t2(z, y, dinv, b1.reshape(1, d), W2)
    z = agg_k(y, src2, dst2, zeros_rd)
    y = t2(z, y, dinv, b2.reshape(1, d), W3)
    z = agg_k(y, src2, dst2, zeros_rd)

    def t3_body(z_ref, y_ref, dinv_ref, b_ref, batch_ref,
                wl1_ref, bl1_ref, wl2_ref, bl2_ref, out_ref):
        dinv = dinv_ref[...]
        h3 = dinv * (z_ref[0, 0:n, :] + z_ref[1, 0:n, :] + y_ref[...]) + b_ref[...]
        gids = lax.broadcasted_iota(I32, (_G, n), 0)
        oh = (batch_ref[...] == gids).astype(F32)
        sums = jnp.dot(oh, h3)
        cnts = jnp.sum(oh, axis=1, keepdims=True)
        pooled = sums / jnp.maximum(cnts, 1.0)
        h2 = jnp.maximum(
            jnp.dot(pooled, wl1_ref[...]) + bl1_ref[...], 0.0)
        out_ref[...] = jnp.dot(h2, wl2_ref[...]) + bl2_ref[...]

    out = pl.pallas_call(
        t3_body, out_shape=jax.ShapeDtypeStruct((_G, 1), F32),
    )(z, y, dinv, b3.reshape(1, d), batch.reshape(1, n),
      Wl1, bl1.reshape(1, -1), Wl2, bl2.reshape(1, 1))
    return out


# speedup vs baseline: 28.4544x; 1.0115x over previous
import functools

import jax
import jax.numpy as jnp
from jax import lax
from jax.experimental import pallas as pl
from jax.experimental.pallas import tpu as pltpu
from jax.experimental.pallas import tpu_sc as plsc

F32 = jnp.float32
I32 = jnp.int32

_NC = 2
_NS = 16
_NW = _NC * _NS
_CHUNK = 128
_G = 64


def _sc_mesh():
    return plsc.VectorSubcoreMesh(core_axis_name="c", subcore_axis_name="s")


def _make_deg_kernel(nch, n_acc, rpt):
    @functools.partial(
        pl.kernel,
        out_type=jax.ShapeDtypeStruct((_NC * n_acc,), F32),
        mesh=_sc_mesh(),
        scratch_types=[
            pltpu.VMEM((nch, _CHUNK), I32),
            pltpu.VMEM((_CHUNK,), F32),
            pltpu.VMEM_SHARED((n_acc,), F32),
            pltpu.SemaphoreType.DMA,
        ],
    )
    def deg_kernel(dst_hbm, zeros_hbm, ones_hbm, out_hbm, dst_v, ones_v, acc, sem):
        cid = lax.axis_index("c")
        sid = lax.axis_index("s")
        wid = cid * _NS + sid
        zero_cp = pltpu.async_copy(zeros_hbm, acc.at[pl.ds(sid * rpt, rpt)],
                                   sem)
        pltpu.sync_copy(dst_hbm.at[pl.ds(wid * nch, nch)], dst_v)
        pltpu.sync_copy(ones_hbm, ones_v)
        zero_cp.wait()
        plsc.subcore_barrier()

        def body(j, carry):
            pltpu.sync_copy(ones_v, acc.at[dst_v.at[j]], add=True)
            return carry

        lax.fori_loop(0, nch, body, 0)
        plsc.subcore_barrier()
        pltpu.sync_copy(acc.at[pl.ds(sid * rpt, rpt)],
                        out_hbm.at[pl.ds(cid * n_acc + sid * rpt, rpt)])

    return deg_kernel


def _make_agg_kernel(nch, n_acc, rpt, d):
    @functools.partial(
        pl.kernel,
        out_type=jax.ShapeDtypeStruct((_NC, n_acc, d), F32),
        mesh=_sc_mesh(),
        scratch_types=[
            pltpu.VMEM((nch // 2, _CHUNK), I32),
            pltpu.VMEM((nch // 2, _CHUNK), I32),
            pltpu.VMEM((_CHUNK, d), F32),
            pltpu.VMEM((_CHUNK, d), F32),
            pltpu.VMEM_SHARED((n_acc, d), F32),
            pltpu.SemaphoreType.DMA,
            pltpu.SemaphoreType.DMA,
        ],
    )
    def agg_kernel(y_hbm, src_hbm, dst_hbm, zeros_hbm, out_hbm,
                   src_v, dst_v, rows0, rows1, acc, sem0, sem1):
        cid = lax.axis_index("c")
        sid = lax.axis_index("s")
        wid = cid * _NS + sid
        nh = nch // 2
        zero_cp = pltpu.async_copy(zeros_hbm, acc.at[pl.ds(sid * rpt, rpt)],
                                   sem1)

        for half in range(2):
            base = wid * nch + half * nh
            pltpu.sync_copy(src_hbm.at[pl.ds(base, nh)], src_v)
            pltpu.sync_copy(dst_hbm.at[pl.ds(base, nh)], dst_v)
            if half == 0:
                zero_cp.wait()
                plsc.subcore_barrier()
            pltpu.async_copy(y_hbm.at[src_v.at[0]], rows0, sem0)

            def body(jj, carry):
                j0 = 2 * jj
                j1 = j0 + 1
                pltpu.async_copy(y_hbm.at[src_v.at[j1]], rows1, sem1)
                pltpu.make_async_copy(y_hbm.at[src_v.at[j0]], rows0,
                                      sem0).wait()
                pltpu.sync_copy(rows0, acc.at[dst_v.at[j0]], add=True)

                @pl.when(j0 + 2 < nh)
                def _():
                    pltpu.async_copy(y_hbm.at[src_v.at[j0 + 2]], rows0, sem0)

                pltpu.make_async_copy(y_hbm.at[src_v.at[j1]], rows1,
                                      sem1).wait()
                pltpu.sync_copy(rows1, acc.at[dst_v.at[j1]], add=True)
                return carry

            lax.fori_loop(0, nh // 2, body, 0)
        plsc.subcore_barrier()
        pltpu.sync_copy(acc.at[pl.ds(sid * rpt, rpt)],
                        out_hbm.at[cid, pl.ds(sid * rpt, rpt)])

    return agg_kernel


def kernel(x, edge_index, batch, W1, b1, W2, b2, W3, b3, Wl1, bl1, Wl2, bl2):
    n, d = x.shape
    e = edge_index.shape[1]

    rpt = ((-(-(n + 1) // _NS)) + 127) // 128 * 128
    n_acc = rpt * _NS
    epc = _NW * _CHUNK
    nch = (-(-e // epc) + 7) // 8 * 8
    e_pad = nch * epc
    pad = e_pad - e

    src = edge_index[0]
    dst = edge_index[1]
    if pad:
        pad_i = jnp.arange(pad, dtype=I32)
        src = jnp.concatenate([src, pad_i % min(n, 1024)])
        dst = jnp.concatenate([dst, n + pad_i % (n_acc - n)])
    src2 = src.reshape(_NW * nch, _CHUNK)
    dst2 = dst.reshape(_NW * nch, _CHUNK)

    zeros_r = jnp.zeros((rpt,), F32)
    zeros_rd = jnp.zeros((rpt, d), F32)
    ones_c = jnp.ones((_CHUNK,), F32)

    deg_k = _make_deg_kernel(nch, n_acc, rpt)
    agg_k = _make_agg_kernel(nch, n_acc, rpt, d)

    deg = deg_k(dst2, zeros_r, ones_c)
    d0 = deg[:n].reshape(n, 1)
    d1 = deg[n_acc:n_acc + n].reshape(n, 1)

    def t1_body(x_ref, w_ref, d0_ref, d1_ref, y_ref, dinv_ref):
        dinv = lax.rsqrt(1.0 + d0_ref[...] + d1_ref[...])
        xw = jnp.dot(x_ref[...], w_ref[...])
        y_ref[...] = dinv * xw
        dinv_ref[...] = dinv

    y, dinv = pl.pallas_call(
        t1_body,
        out_shape=(jax.ShapeDtypeStruct((n, d), F32),
                   jax.ShapeDtypeStruct((n, 1), F32)),
    )(x, W1, d0, d1)

    def t2_body(z_ref, y_ref, dinv_ref, b_ref, w_ref, yn_ref):
        dinv = dinv_ref[...]
        agg = z_ref[0, 0:n, :] + z_ref[1, 0:n, :] + y_ref[...]
        h = jnp.maximum(dinv * agg + b_ref[...], 0.0)
        yn_ref[...] = dinv * jnp.dot(h, w_ref[...])

    t2 = pl.pallas_call(t2_body, out_shape=jax.ShapeDtypeStruct((n, d), F32))

    z = agg_k(y, src2, dst2, zeros_rd)
    y = t2(z, y, dinv, b1.reshape(1, d), W2)
    z = agg_k(y, src2, dst2, zeros_rd)
    y = t2(z, y, dinv, b2.reshape(1, d), W3)
    z = agg_k(y, src2, dst2, zeros_rd)

    def t3_body(z_ref, y_ref, dinv_ref, b_ref, batch_ref,
                wl1_ref, bl1_ref, wl2_ref, bl2_ref, out_ref):
        dinv = dinv_ref[...]
        h3 = dinv * (z_ref[0, 0:n, :] + z_ref[1, 0:n, :] + y_ref[...]) + b_ref[...]
        gids = lax.broadcasted_iota(I32, (_G, n), 0)
        oh = (batch_ref[...] == gids).astype(F32)
        sums = jnp.dot(oh, h3)
        cnts = jnp.sum(oh, axis=1, keepdims=True)
        pooled = sums / jnp.maximum(cnts, 1.0)
        h2 = jnp.maximum(
            jnp.dot(pooled, wl1_ref[...]) + bl1_ref[...], 0.0)
        out_ref[...] = jnp.dot(h2, wl2_ref[...]) + bl2_ref[...]

    out = pl.pallas_call(
        t3_body, out_shape=jax.ShapeDtypeStruct((_G, 1), F32),
    )(z, y, dinv, b3.reshape(1, d), batch.reshape(1, n),
      Wl1, bl1.reshape(1, -1), Wl2, bl2.reshape(1, 1))
    return out
